# Initial kernel scaffold; baseline (speedup 1.0000x reference)
import jax
import jax.numpy as jnp
from jax.experimental import pallas as pl


def kernel(perturbation_mask, edge_index, emb_weight, W1, b1, W2, b2, head_W, head_b):
    raise NotImplementedError("write your pallas kernel here")


# speedup vs baseline: 8.2793x; 8.2793x over previous
import jax
import jax.numpy as jnp
from jax import lax
from jax.experimental import pallas as pl
from jax.experimental.pallas import tpu as pltpu
from jax.experimental.pallas import tpu_sc as plsc

N = 10000
E = 160000
D = 256
DH = 128
NC = 2
NS = 16
CH = 128
E_PAD = 163840
DUMP = N
NP = 10240
RPT = NP // NS
AW = 128
R = NP // 16
GRID = NP // R

_mesh = plsc.VectorSubcoreMesh(core_axis_name="c", subcore_axis_name="s")


def _deg_body(dst_hbm, ones_hbm, zeros_hbm, out_hbm, dst_v, ones_v, acc, sem):
  cid = lax.axis_index("c")
  sid = lax.axis_index("s")
  pltpu.sync_copy(zeros_hbm, acc.at[pl.ds(sid * RPT, RPT)])
  pltpu.sync_copy(ones_hbm, ones_v)
  plsc.subcore_barrier()
  per_tile = E_PAD // (NC * NS)
  tbase = (cid * NS + sid) * per_tile
  nchunks = per_tile // CH

  def body(i, carry):
    base = tbase + i * CH
    pltpu.sync_copy(dst_hbm.at[pl.ds(base, CH)], dst_v)
    pltpu.sync_copy(ones_v, acc.at[dst_v], add=True)
    return carry

  lax.fori_loop(0, nchunks, body, 0)
  plsc.subcore_barrier()
  pltpu.sync_copy(acc.at[pl.ds(sid * RPT, RPT)],
                  out_hbm.at[cid, pl.ds(sid * RPT, RPT)])


_deg_call = pl.kernel(
    _deg_body,
    out_type=jax.ShapeDtypeStruct((NC, NP, AW), jnp.float32),
    mesh=_mesh,
    scratch_types=[
        pltpu.VMEM((CH,), jnp.int32),
        pltpu.VMEM((CH, AW), jnp.float32),
        pltpu.VMEM_SHARED((NP, AW), jnp.float32),
        pltpu.SemaphoreType.DMA,
    ],
)


def _agg_body(src_hbm, dst_hbm, y0_hbm, y1_hbm, z0_hbm, z1_hbm,
              src_v, dst_v, rows_v, acc, sem):
  cid = lax.axis_index("c")
  sid = lax.axis_index("s")
  per_tile = E_PAD // NS
  tbase = sid * per_tile
  nchunks = per_tile // CH

  def run_half(y_hbm, z_hbm):
    pltpu.sync_copy(y_hbm.at[pl.ds(sid * RPT, RPT)],
                    acc.at[pl.ds(sid * RPT, RPT)])
    plsc.subcore_barrier()

    def body(i, carry):
      base = tbase + i * CH
      pltpu.sync_copy(src_hbm.at[pl.ds(base, CH)], src_v)
      pltpu.sync_copy(dst_hbm.at[pl.ds(base, CH)], dst_v)
      pltpu.async_copy(y_hbm.at[src_v], rows_v, sem).wait()
      pltpu.sync_copy(rows_v, acc.at[dst_v], add=True)
      return carry

    lax.fori_loop(0, nchunks, body, 0)
    plsc.subcore_barrier()
    pltpu.sync_copy(acc.at[pl.ds(sid * RPT, RPT)],
                    z_hbm.at[pl.ds(sid * RPT, RPT)])

  @pl.when(cid == 0)
  def _():
    run_half(y0_hbm, z0_hbm)

  @pl.when(cid == 1)
  def _():
    run_half(y1_hbm, z1_hbm)


_agg_call = pl.kernel(
    _agg_body,
    out_type=(
        jax.ShapeDtypeStruct((NP, DH), jnp.float32),
        jax.ShapeDtypeStruct((NP, DH), jnp.float32),
    ),
    mesh=_mesh,
    scratch_types=[
        pltpu.VMEM((CH,), jnp.int32),
        pltpu.VMEM((CH,), jnp.int32),
        pltpu.VMEM((CH, DH), jnp.float32),
        pltpu.VMEM_SHARED((NP, DH), jnp.float32),
        pltpu.SemaphoreType.DMA,
    ],
)


def _sagg_body(src_hbm, dst_hbm, t_hbm, zeros_hbm, u0_hbm, u1_hbm,
               src_v, dst_v, rows_v, acc, sem):
  cid = lax.axis_index("c")
  sid = lax.axis_index("s")

  @pl.when(cid == 0)
  def _():
    pltpu.sync_copy(t_hbm.at[pl.ds(sid * RPT, RPT)],
                    acc.at[pl.ds(sid * RPT, RPT)])

  @pl.when(cid == 1)
  def _():
    pltpu.sync_copy(zeros_hbm, acc.at[pl.ds(sid * RPT, RPT)])

  plsc.subcore_barrier()
  per_tile = E_PAD // (NC * NS)
  tbase = (cid * NS + sid) * per_tile
  nchunks = per_tile // CH

  def body(i, carry):
    base = tbase + i * CH
    pltpu.sync_copy(src_hbm.at[pl.ds(base, CH)], src_v)
    pltpu.sync_copy(dst_hbm.at[pl.ds(base, CH)], dst_v)
    pltpu.async_copy(t_hbm.at[src_v], rows_v, sem).wait()
    pltpu.sync_copy(rows_v, acc.at[dst_v], add=True)
    return carry

  lax.fori_loop(0, nchunks, body, 0)
  plsc.subcore_barrier()

  @pl.when(cid == 0)
  def _():
    pltpu.sync_copy(acc.at[pl.ds(sid * RPT, RPT)],
                    u0_hbm.at[pl.ds(sid * RPT, RPT)])

  @pl.when(cid == 1)
  def _():
    pltpu.sync_copy(acc.at[pl.ds(sid * RPT, RPT)],
                    u1_hbm.at[pl.ds(sid * RPT, RPT)])


_sagg_call = pl.kernel(
    _sagg_body,
    out_type=(
        jax.ShapeDtypeStruct((NP, AW), jnp.float32),
        jax.ShapeDtypeStruct((NP, AW), jnp.float32),
    ),
    mesh=_mesh,
    scratch_types=[
        pltpu.VMEM((CH,), jnp.int32),
        pltpu.VMEM((CH,), jnp.int32),
        pltpu.VMEM((CH, AW), jnp.float32),
        pltpu.VMEM_SHARED((NP, AW), jnp.float32),
        pltpu.SemaphoreType.DMA,
    ],
)


def _dinv_from(degacc_blk):
  deg = degacc_blk[0, :, 0:1] + degacc_blk[1, :, 0:1] + 1.0
  return lax.rsqrt(deg)


def _scale_body(emb_ref, mask_ref, degacc_ref, y0_ref, y1_ref):
  dinv = _dinv_from(degacc_ref[...])
  y = emb_ref[...] * (mask_ref[...] * dinv)
  y0_ref[...] = y[:, :DH]
  y1_ref[...] = y[:, DH:]


def _dense_body(z0_ref, z1_ref, degacc_ref, w1_ref, b1_ref, w2_ref, hw_ref,
                t_ref):
  dinv = _dinv_from(degacc_ref[...])
  a = jnp.concatenate([z0_ref[...], z1_ref[...]], axis=1) * dinv
  h = lax.dot_general(a, w1_ref[...], (((1,), (0,)), ((), ())),
                      preferred_element_type=jnp.float32) + b1_ref[...]
  g = 0.5 * h * (1.0 + lax.erf(h * 0.7071067811865476))
  w = lax.dot_general(w2_ref[...], hw_ref[...], (((1,), (0,)), ((), ())),
                      preferred_element_type=jnp.float32)
  s = lax.dot_general(g, w, (((1,), (0,)), ((), ())),
                      preferred_element_type=jnp.float32)
  t_ref[...] = jnp.broadcast_to(dinv * s, (t_ref.shape[0], AW))


def _final_body(u0_ref, u1_ref, degacc_ref, b2_ref, hwt_ref, hb_ref, out_ref):
  dinv = _dinv_from(degacc_ref[...])
  c0 = jnp.sum(b2_ref[...] * hwt_ref[...]) + hb_ref[0, 0]
  out_ref[...] = dinv * (u0_ref[:, 0:1] + u1_ref[:, 0:1]) + c0


def _deg_spec():
  return pl.BlockSpec((NC, R, AW), lambda i: (0, i, 0))


_scale_call = pl.pallas_call(
    _scale_body,
    grid=(GRID,),
    in_specs=[
        pl.BlockSpec((R, D), lambda i: (i, 0)),
        pl.BlockSpec((R, 1), lambda i: (i, 0)),
        _deg_spec(),
    ],
    out_specs=[
        pl.BlockSpec((R, DH), lambda i: (i, 0)),
        pl.BlockSpec((R, DH), lambda i: (i, 0)),
    ],
    out_shape=[
        jax.ShapeDtypeStruct((NP, DH), jnp.float32),
        jax.ShapeDtypeStruct((NP, DH), jnp.float32),
    ],
)

_dense_call = pl.pallas_call(
    _dense_body,
    grid=(GRID,),
    in_specs=[
        pl.BlockSpec((R, DH), lambda i: (i, 0)),
        pl.BlockSpec((R, DH), lambda i: (i, 0)),
        _deg_spec(),
        pl.BlockSpec((D, 2 * D), lambda i: (0, 0)),
        pl.BlockSpec((1, 2 * D), lambda i: (0, 0)),
        pl.BlockSpec((2 * D, D), lambda i: (0, 0)),
        pl.BlockSpec((D, 1), lambda i: (0, 0)),
    ],
    out_specs=pl.BlockSpec((R, AW), lambda i: (i, 0)),
    out_shape=jax.ShapeDtypeStruct((NP, AW), jnp.float32),
)

_final_call = pl.pallas_call(
    _final_body,
    grid=(GRID,),
    in_specs=[
        pl.BlockSpec((R, AW), lambda i: (i, 0)),
        pl.BlockSpec((R, AW), lambda i: (i, 0)),
        _deg_spec(),
        pl.BlockSpec((1, D), lambda i: (0, 0)),
        pl.BlockSpec((1, D), lambda i: (0, 0)),
        pl.BlockSpec((1, 1), lambda i: (0, 0)),
    ],
    out_specs=pl.BlockSpec((R, 1), lambda i: (i, 0)),
    out_shape=jax.ShapeDtypeStruct((NP, 1), jnp.float32),
)


@jax.jit
def kernel(perturbation_mask, edge_index, emb_weight, W1, b1, W2, b2,
           head_W, head_b):
  src = edge_index[0]
  dst = edge_index[1]
  npad = E_PAD - E
  src_pad = jnp.concatenate([src, jnp.zeros((npad,), jnp.int32)])
  dst_pad = jnp.concatenate([dst, jnp.full((npad,), DUMP, jnp.int32)])
  ones16 = jnp.ones((CH, AW), jnp.float32)
  zeros16 = jnp.zeros((RPT, AW), jnp.float32)
  emb_pad = jnp.pad(emb_weight, ((0, NP - N), (0, 0)))
  mask_pad = jnp.pad(perturbation_mask[:, None], ((0, NP - N), (0, 0)))

  degacc = _deg_call(dst_pad, ones16, zeros16)
  y0, y1 = _scale_call(emb_pad, mask_pad, degacc)
  z0, z1 = _agg_call(src_pad, dst_pad, y0, y1)
  t_pad = _dense_call(z0, z1, degacc, W1, b1[None, :], W2, head_W)
  u0, u1 = _sagg_call(src_pad, dst_pad, t_pad, zeros16)
  out = _final_call(u0, u1, degacc, b2[None, :], head_W[:, 0][None, :],
                    head_b[None, :])
  return out[:N, 0]


# speedup vs baseline: 9.9923x; 1.2069x over previous
import jax
import jax.numpy as jnp
from jax import lax
from jax.experimental import pallas as pl
from jax.experimental.pallas import tpu as pltpu
from jax.experimental.pallas import tpu_sc as plsc

N = 10000
E = 160000
D = 256
DH = 128
NC = 2
NS = 16
CH = 128
E_PAD = 163840
ECH = E_PAD // CH
DUMP = N
NP = 10240
RPT = NP // NS
AW = 128
C3 = ECH // NS
C1 = ECH // (NC * NS)
R = NP // 16
GRID = NP // R

_mesh = plsc.VectorSubcoreMesh(core_axis_name="c", subcore_axis_name="s")


def _deg_body(dst_hbm, ones_hbm, zeros_hbm, out_hbm, idx_v, ones_v, acc, sem):
  cid = lax.axis_index("c")
  sid = lax.axis_index("s")
  pltpu.sync_copy(zeros_hbm, acc.at[pl.ds(sid * RPT, RPT)])
  pltpu.sync_copy(ones_hbm, ones_v)
  pltpu.sync_copy(dst_hbm.at[pl.ds((cid * NS + sid) * C1, C1)], idx_v)
  plsc.subcore_barrier()

  def fire(i, carry):
    pltpu.async_copy(ones_v, acc.at[idx_v.at[i]], sem, add=True)
    return carry

  lax.fori_loop(0, C1, fire, 0)

  def drain(i, carry):
    pltpu.make_async_copy(ones_v, acc.at[idx_v.at[0]], sem).wait()
    return carry

  lax.fori_loop(0, C1, drain, 0)
  plsc.subcore_barrier()
  pltpu.sync_copy(acc.at[pl.ds(sid * RPT, RPT)],
                  out_hbm.at[cid, pl.ds(sid * RPT, RPT)])


_deg_call = pl.kernel(
    _deg_body,
    out_type=jax.ShapeDtypeStruct((NC, NP, AW), jnp.float32),
    mesh=_mesh,
    scratch_types=[
        pltpu.VMEM((C1, CH), jnp.int32),
        pltpu.VMEM((CH, AW), jnp.float32),
        pltpu.VMEM_SHARED((NP, AW), jnp.float32),
        pltpu.SemaphoreType.DMA,
    ],
)


def _agg_body(src_hbm, dst_hbm, y0_hbm, y1_hbm, z0_hbm, z1_hbm,
              src_b, dst_b, rows_v, acc, sems):
  cid = lax.axis_index("c")
  sid = lax.axis_index("s")

  def run_half(y_hbm, z_hbm):
    pltpu.sync_copy(y_hbm.at[pl.ds(sid * RPT, RPT)],
                    acc.at[pl.ds(sid * RPT, RPT)])
    plsc.subcore_barrier()
    cb = sid * C3

    def idx_load(i, b):
      pltpu.sync_copy(src_hbm.at[cb + i], src_b.at[b])
      pltpu.sync_copy(dst_hbm.at[cb + i], dst_b.at[b])

    def gather_start(b):
      pltpu.async_copy(y_hbm.at[src_b.at[b]], rows_v.at[b], sems.at[b])

    def gather_wait(b):
      pltpu.make_async_copy(y_hbm.at[src_b.at[b]], rows_v.at[b],
                            sems.at[b]).wait()

    def scatter(b):
      pltpu.sync_copy(rows_v.at[b], acc.at[dst_b.at[b]], add=True)

    idx_load(0, 0)
    gather_start(0)

    def pair(g, carry):
      i0 = 2 * g
      idx_load(i0 + 1, 1)
      gather_start(1)
      gather_wait(0)
      scatter(0)

      @pl.when(g < C3 // 2 - 1)
      def _():
        idx_load(i0 + 2, 0)
        gather_start(0)

      gather_wait(1)
      scatter(1)
      return carry

    lax.fori_loop(0, C3 // 2, pair, 0)
    plsc.subcore_barrier()
    pltpu.sync_copy(acc.at[pl.ds(sid * RPT, RPT)],
                    z_hbm.at[pl.ds(sid * RPT, RPT)])

  @pl.when(cid == 0)
  def _():
    run_half(y0_hbm, z0_hbm)

  @pl.when(cid == 1)
  def _():
    run_half(y1_hbm, z1_hbm)


_agg_call = pl.kernel(
    _agg_body,
    out_type=(
        jax.ShapeDtypeStruct((NP, DH), jnp.float32),
        jax.ShapeDtypeStruct((NP, DH), jnp.float32),
    ),
    mesh=_mesh,
    scratch_types=[
        pltpu.VMEM((2, CH), jnp.int32),
        pltpu.VMEM((2, CH), jnp.int32),
        pltpu.VMEM((2, CH, DH), jnp.float32),
        pltpu.VMEM_SHARED((NP, DH), jnp.float32),
        pltpu.SemaphoreType.DMA((2,)),
    ],
)


def _sagg_body(src_hbm, dst_hbm, t_hbm, zeros_hbm, u0_hbm, u1_hbm,
               src_b, dst_b, rows_v, acc, sems):
  cid = lax.axis_index("c")
  sid = lax.axis_index("s")

  def run(seed_t, cb, u_hbm):
    if seed_t:
      pltpu.sync_copy(t_hbm.at[pl.ds(sid * RPT, RPT)],
                      acc.at[pl.ds(sid * RPT, RPT)])
    else:
      pltpu.sync_copy(zeros_hbm, acc.at[pl.ds(sid * RPT, RPT)])
    plsc.subcore_barrier()

    def idx_load(i, b):
      pltpu.sync_copy(src_hbm.at[cb + i], src_b.at[b])
      pltpu.sync_copy(dst_hbm.at[cb + i], dst_b.at[b])

    def gather_start(b):
      pltpu.async_copy(t_hbm.at[src_b.at[b]], rows_v.at[b], sems.at[b])

    def gather_wait(b):
      pltpu.make_async_copy(t_hbm.at[src_b.at[b]], rows_v.at[b],
                            sems.at[b]).wait()

    def scatter(b):
      pltpu.sync_copy(rows_v.at[b], acc.at[dst_b.at[b]], add=True)

    idx_load(0, 0)
    gather_start(0)

    def pair(g, carry):
      i0 = 2 * g
      idx_load(i0 + 1, 1)
      gather_start(1)
      gather_wait(0)
      scatter(0)

      @pl.when(g < C1 // 2 - 1)
      def _():
        idx_load(i0 + 2, 0)
        gather_start(0)

      gather_wait(1)
      scatter(1)
      return carry

    lax.fori_loop(0, C1 // 2, pair, 0)
    plsc.subcore_barrier()
    pltpu.sync_copy(acc.at[pl.ds(sid * RPT, RPT)],
                    u_hbm.at[pl.ds(sid * RPT, RPT)])

  @pl.when(cid == 0)
  def _():
    run(True, sid * C1, u0_hbm)

  @pl.when(cid == 1)
  def _():
    run(False, (NS + sid) * C1, u1_hbm)


_sagg_call = pl.kernel(
    _sagg_body,
    out_type=(
        jax.ShapeDtypeStruct((NP, AW), jnp.float32),
        jax.ShapeDtypeStruct((NP, AW), jnp.float32),
    ),
    mesh=_mesh,
    scratch_types=[
        pltpu.VMEM((2, CH), jnp.int32),
        pltpu.VMEM((2, CH), jnp.int32),
        pltpu.VMEM((2, CH, AW), jnp.float32),
        pltpu.VMEM_SHARED((NP, AW), jnp.float32),
        pltpu.SemaphoreType.DMA((2,)),
    ],
)


def _scale_body(emb_ref, mask_ref, degacc_ref, y0_ref, y1_ref, dinv_ref):
  deg = degacc_ref[0, :, 0:1] + degacc_ref[1, :, 0:1] + 1.0
  dinv = lax.rsqrt(deg)
  y = emb_ref[...] * (mask_ref[...] * dinv)
  y0_ref[...] = y[:, :DH]
  y1_ref[...] = y[:, DH:]
  dinv_ref[...] = jnp.broadcast_to(dinv, dinv_ref.shape)


def _dense_body(z0_ref, z1_ref, dinv_ref, w1_ref, b1_ref, w2_ref, hw_ref,
                t_ref):
  dinv = dinv_ref[:, 0:1]
  a = jnp.concatenate([z0_ref[...], z1_ref[...]], axis=1) * dinv
  h = lax.dot_general(a, w1_ref[...], (((1,), (0,)), ((), ())),
                      preferred_element_type=jnp.float32) + b1_ref[...]
  g = 0.5 * h * (1.0 + lax.erf(h * 0.7071067811865476))
  w = lax.dot_general(w2_ref[...], hw_ref[...], (((1,), (0,)), ((), ())),
                      preferred_element_type=jnp.float32)
  s = lax.dot_general(g, w, (((1,), (0,)), ((), ())),
                      preferred_element_type=jnp.float32)
  t_ref[...] = jnp.broadcast_to(dinv * s, (t_ref.shape[0], AW))


def _final_body(u0_ref, u1_ref, dinv_ref, b2_ref, hwt_ref, hb_ref, out_ref):
  c0 = jnp.sum(b2_ref[...] * hwt_ref[...]) + hb_ref[0, 0]
  out_ref[...] = dinv_ref[:, 0:1] * (u0_ref[:, 0:1] + u1_ref[:, 0:1]) + c0


_scale_call = pl.pallas_call(
    _scale_body,
    grid=(GRID,),
    in_specs=[
        pl.BlockSpec((R, D), lambda i: (i, 0)),
        pl.BlockSpec((R, 1), lambda i: (i, 0)),
        pl.BlockSpec((NC, R, AW), lambda i: (0, i, 0)),
    ],
    out_specs=[
        pl.BlockSpec((R, DH), lambda i: (i, 0)),
        pl.BlockSpec((R, DH), lambda i: (i, 0)),
        pl.BlockSpec((R, AW), lambda i: (i, 0)),
    ],
    out_shape=[
        jax.ShapeDtypeStruct((NP, DH), jnp.float32),
        jax.ShapeDtypeStruct((NP, DH), jnp.float32),
        jax.ShapeDtypeStruct((NP, AW), jnp.float32),
    ],
)

_dense_call = pl.pallas_call(
    _dense_body,
    grid=(GRID,),
    in_specs=[
        pl.BlockSpec((R, DH), lambda i: (i, 0)),
        pl.BlockSpec((R, DH), lambda i: (i, 0)),
        pl.BlockSpec((R, AW), lambda i: (i, 0)),
        pl.BlockSpec((D, 2 * D), lambda i: (0, 0)),
        pl.BlockSpec((1, 2 * D), lambda i: (0, 0)),
        pl.BlockSpec((2 * D, D), lambda i: (0, 0)),
        pl.BlockSpec((D, 1), lambda i: (0, 0)),
    ],
    out_specs=pl.BlockSpec((R, AW), lambda i: (i, 0)),
    out_shape=jax.ShapeDtypeStruct((NP, AW), jnp.float32),
)

_final_call = pl.pallas_call(
    _final_body,
    grid=(GRID,),
    in_specs=[
        pl.BlockSpec((R, AW), lambda i: (i, 0)),
        pl.BlockSpec((R, AW), lambda i: (i, 0)),
        pl.BlockSpec((R, AW), lambda i: (i, 0)),
        pl.BlockSpec((1, D), lambda i: (0, 0)),
        pl.BlockSpec((1, D), lambda i: (0, 0)),
        pl.BlockSpec((1, 1), lambda i: (0, 0)),
    ],
    out_specs=pl.BlockSpec((R, 1), lambda i: (i, 0)),
    out_shape=jax.ShapeDtypeStruct((NP, 1), jnp.float32),
)


@jax.jit
def kernel(perturbation_mask, edge_index, emb_weight, W1, b1, W2, b2,
           head_W, head_b):
  src = edge_index[0]
  dst = edge_index[1]
  npad = E_PAD - E
  src2 = jnp.concatenate([src, jnp.zeros((npad,), jnp.int32)]).reshape(ECH, CH)
  dst2 = jnp.concatenate([dst, jnp.full((npad,), DUMP, jnp.int32)]
                         ).reshape(ECH, CH)
  ones128 = jnp.ones((CH, AW), jnp.float32)
  zeros128 = jnp.zeros((RPT, AW), jnp.float32)
  emb_pad = jnp.pad(emb_weight, ((0, NP - N), (0, 0)))
  mask_pad = jnp.pad(perturbation_mask[:, None], ((0, NP - N), (0, 0)))

  degacc = _deg_call(dst2, ones128, zeros128)
  y0, y1, dinv = _scale_call(emb_pad, mask_pad, degacc)
  z0, z1 = _agg_call(src2, dst2, y0, y1)
  t_pad = _dense_call(z0, z1, dinv, W1, b1[None, :], W2, head_W)
  u0, u1 = _sagg_call(src2, dst2, t_pad, zeros128)
  out = _final_call(u0, u1, dinv, b2[None, :], head_W[:, 0][None, :],
                    head_b[None, :])
  return out[:N, 0]


# speedup vs baseline: 16.1480x; 1.6160x over previous
import jax
import jax.numpy as jnp
from jax import lax
from jax.experimental import pallas as pl
from jax.experimental.pallas import tpu as pltpu
from jax.experimental.pallas import tpu_sc as plsc

N = 10000
E = 160000
D = 256
DH = 128
NC = 2
NS = 16
CH = 80
E_PAD = 163840
ECH = E_PAD // CH
DUMP = N
NP = 10240
RPT = NP // NS
AW = 128
C3 = ECH // NS
C1 = ECH // (NC * NS)
R = NP // 16
GRID = NP // R
EPT = E_PAD // (NC * NS)
VL = 16

_mesh = plsc.VectorSubcoreMesh(core_axis_name="c", subcore_axis_name="s")


def _edge_pipe(src_hbm, dst_hbm, tab, acc, src_b, dst_b, rows_v,
               sem_i, sem_g, sem_s, cb, n):
  def idx_start(i, b):
    pltpu.async_copy(src_hbm.at[cb + i], src_b.at[b], sem_i.at[b])
    pltpu.async_copy(dst_hbm.at[cb + i], dst_b.at[b], sem_i.at[b])

  def idx_wait(i, b):
    pltpu.make_async_copy(src_hbm.at[cb + i], src_b.at[b], sem_i.at[b]).wait()
    pltpu.make_async_copy(dst_hbm.at[cb + i], dst_b.at[b], sem_i.at[b]).wait()

  def g_start(bi, br):
    pltpu.async_copy(tab.at[src_b.at[bi]], rows_v.at[br], sem_g.at[br])

  def g_wait(bi, br):
    pltpu.make_async_copy(tab.at[src_b.at[bi]], rows_v.at[br],
                          sem_g.at[br]).wait()

  def s_start(bi, br):
    pltpu.async_copy(rows_v.at[br], acc.at[dst_b.at[bi]], sem_s.at[br],
                     add=True)

  def s_wait(bi, br):
    pltpu.make_async_copy(rows_v.at[br], acc.at[dst_b.at[bi]],
                          sem_s.at[br]).wait()

  idx_start(0, 0)
  idx_start(1, 1)
  idx_start(2, 2)
  idx_wait(0, 0)
  g_start(0, 0)
  idx_wait(1, 1)
  g_start(1, 1)
  groups = n // 8

  def group(gi, carry):
    for j in range(8):
      i = 8 * gi + j
      bi = j
      br = i % 4 if False else j % 4
      bi2 = (j + 2) % 8
      br2 = (j + 2) % 4
      bi3 = (j + 3) % 8
      if j >= 2:
        s_wait((j - 2) % 8, (j - 2) % 4)
      else:

        @pl.when(gi > 0)
        def _(a=(j - 2) % 8, b=(j - 2) % 4):
          s_wait(a, b)

      if j < 6:
        idx_wait(i + 2, bi2)
        g_start(bi2, br2)
      else:

        @pl.when(gi < groups - 1)
        def _():
          idx_wait(i + 2, bi2)
          g_start(bi2, br2)

      if j < 5:
        idx_start(i + 3, bi3)
      else:

        @pl.when(gi < groups - 1)
        def _():
          idx_start(i + 3, bi3)

      g_wait(bi, br)
      s_start(bi, br)
    return carry

  lax.fori_loop(0, groups, group, 0)
  s_wait(6, 2)
  s_wait(7, 3)


def _hist_reduce(acc_v, stage, red_v, res_v, out_hbm, cid, sid):
  pltpu.sync_copy(acc_v, stage.at[sid])
  plsc.subcore_barrier()
  for r in range(NS):
    pltpu.sync_copy(stage.at[r, pl.ds(sid * RPT, RPT)],
                    red_v.at[pl.ds(r * RPT, RPT)])

  def red_body(v, carry):
    x = red_v[pl.ds(v * VL, VL)]
    for r in range(1, NS):
      x = x + red_v[pl.ds(r * RPT + v * VL, VL)]
    res_v[pl.ds(v * VL, VL)] = x
    return carry

  lax.fori_loop(0, RPT // VL, red_body, 0)
  pltpu.sync_copy(res_v, out_hbm.at[cid, pl.ds(sid * RPT, RPT)])


def _deg_body(dst_hbm, out_hbm, dst_v, acc_v, red_v, res_v, stage):
  cid = lax.axis_index("c")
  sid = lax.axis_index("s")
  pltpu.sync_copy(dst_hbm.at[pl.ds((cid * NS + sid) * EPT, EPT)], dst_v)

  def zero(r, carry):
    acc_v[pl.ds(r * VL, VL)] = jnp.zeros((VL,), jnp.float32)
    return carry

  lax.fori_loop(0, NP // VL, zero, 0)
  ones = jnp.ones((VL,), jnp.float32)

  def body(k, carry):
    d16 = dst_v[pl.ds(k * VL, VL)]
    plsc.addupdate_scatter(acc_v, [d16], ones)
    return carry

  lax.fori_loop(0, EPT // VL, body, 0)
  _hist_reduce(acc_v, stage, red_v, res_v, out_hbm, cid, sid)


_deg_call = pl.kernel(
    _deg_body,
    out_type=jax.ShapeDtypeStruct((NC, NP), jnp.float32),
    mesh=_mesh,
    compiler_params=pltpu.CompilerParams(needs_layout_passes=False),
    scratch_types=[
        pltpu.VMEM((EPT,), jnp.int32),
        pltpu.VMEM((NP,), jnp.float32),
        pltpu.VMEM((NS * RPT,), jnp.float32),
        pltpu.VMEM((RPT,), jnp.float32),
        pltpu.VMEM_SHARED((NS, NP), jnp.float32),
    ],
)


def _agg_body(src_hbm, dst_hbm, y_hbm, z_hbm,
              src_b, dst_b, rows_v, acc, sem_i, sem_g, sem_s):
  cid = lax.axis_index("c")
  sid = lax.axis_index("s")
  pltpu.sync_copy(y_hbm.at[cid, pl.ds(sid * RPT, RPT)],
                  acc.at[pl.ds(sid * RPT, RPT)])
  plsc.subcore_barrier()
  _edge_pipe(src_hbm, dst_hbm, y_hbm.at[cid], acc, src_b, dst_b, rows_v,
             sem_i, sem_g, sem_s, sid * C3, C3)
  plsc.subcore_barrier()
  pltpu.sync_copy(acc.at[pl.ds(sid * RPT, RPT)],
                  z_hbm.at[cid, pl.ds(sid * RPT, RPT)])


_agg_call = pl.kernel(
    _agg_body,
    out_type=jax.ShapeDtypeStruct((NC, NP, DH), jnp.float32),
    mesh=_mesh,
    scratch_types=[
        pltpu.VMEM((8, CH), jnp.int32),
        pltpu.VMEM((8, CH), jnp.int32),
        pltpu.VMEM((4, CH, DH), jnp.float32),
        pltpu.VMEM_SHARED((NP, DH), jnp.float32),
        pltpu.SemaphoreType.DMA((8,)),
        pltpu.SemaphoreType.DMA((4,)),
        pltpu.SemaphoreType.DMA((4,)),
    ],
)


def _sagg_body(src_hbm, dst_hbm, t8_hbm, u_hbm,
               src_v, dst_v, t_v, acc_v, red_v, res_v, stage):
  cid = lax.axis_index("c")
  sid = lax.axis_index("s")
  eb = (cid * NS + sid) * EPT
  pltpu.sync_copy(t8_hbm.at[0], t_v)
  pltpu.sync_copy(src_hbm.at[pl.ds(eb, EPT)], src_v)
  pltpu.sync_copy(dst_hbm.at[pl.ds(eb, EPT)], dst_v)

  first = jnp.logical_and(cid == 0, sid == 0)

  @pl.when(first)
  def _():
    pltpu.sync_copy(t8_hbm.at[0], acc_v)

  @pl.when(jnp.logical_not(first))
  def _():

    def zero(r, carry):
      acc_v[pl.ds(r * VL, VL)] = jnp.zeros((VL,), jnp.float32)
      return carry

    lax.fori_loop(0, NP // VL, zero, 0)

  def body(k, carry):
    sl = pl.ds(k * VL, VL)
    s16 = src_v[sl]
    d16 = dst_v[sl]
    vals = plsc.load_gather(t_v, [s16])
    plsc.addupdate_scatter(acc_v, [d16], vals)
    return carry

  lax.fori_loop(0, EPT // VL, body, 0)
  _hist_reduce(acc_v, stage, red_v, res_v, u_hbm, cid, sid)


_sagg_call = pl.kernel(
    _sagg_body,
    out_type=jax.ShapeDtypeStruct((NC, NP), jnp.float32),
    mesh=_mesh,
    compiler_params=pltpu.CompilerParams(needs_layout_passes=False),
    scratch_types=[
        pltpu.VMEM((EPT,), jnp.int32),
        pltpu.VMEM((EPT,), jnp.int32),
        pltpu.VMEM((NP,), jnp.float32),
        pltpu.VMEM((NP,), jnp.float32),
        pltpu.VMEM((NS * RPT,), jnp.float32),
        pltpu.VMEM((RPT,), jnp.float32),
        pltpu.VMEM_SHARED((NS, NP), jnp.float32),
    ],
)


def _scale_body(emb_ref, mask_ref, degacc_ref, y_ref, dinv_ref):
  deg = degacc_ref[0] + degacc_ref[1] + 1.0
  dinv = lax.rsqrt(deg)
  y = emb_ref[...] * (mask_ref[...] * dinv)
  y_ref[0] = y[:, :DH]
  y_ref[1] = y[:, DH:]
  dinv_ref[...] = jnp.broadcast_to(dinv, dinv_ref.shape)


def _dense_body(z_ref, dinv_ref, w1_ref, b1_ref, w2_ref, hw_ref,
                t8_ref):
  dinv = dinv_ref[:, 0:1]
  a = jnp.concatenate([z_ref[0], z_ref[1]], axis=1) * dinv
  h = lax.dot_general(a, w1_ref[...], (((1,), (0,)), ((), ())),
                      preferred_element_type=jnp.float32) + b1_ref[...]
  g = 0.5 * h * (1.0 + lax.erf(h * 0.7071067811865476))
  w = lax.dot_general(w2_ref[...], hw_ref[...], (((1,), (0,)), ((), ())),
                      preferred_element_type=jnp.float32)
  s = lax.dot_general(g, w, (((1,), (0,)), ((), ())),
                      preferred_element_type=jnp.float32)
  t = dinv * s
  t_row = jnp.transpose(t, (1, 0))
  t8_ref[...] = jnp.broadcast_to(t_row, (8, t_row.shape[1]))


def _final_body(u_ref, dinv_ref, b2_ref, hwt_ref, hb_ref, out_ref):
  c0 = jnp.sum(b2_ref[...] * hwt_ref[...]) + hb_ref[0, 0]
  out_ref[...] = dinv_ref[:, 0:1] * (u_ref[0] + u_ref[1]) + c0


_scale_call = pl.pallas_call(
    _scale_body,
    grid=(GRID,),
    in_specs=[
        pl.BlockSpec((R, D), lambda i: (i, 0)),
        pl.BlockSpec((R, 1), lambda i: (i, 0)),
        pl.BlockSpec((NC, R, 1), lambda i: (0, i, 0)),
    ],
    out_specs=[
        pl.BlockSpec((NC, R, DH), lambda i: (0, i, 0)),
        pl.BlockSpec((R, AW), lambda i: (i, 0)),
    ],
    out_shape=[
        jax.ShapeDtypeStruct((NC, NP, DH), jnp.float32),
        jax.ShapeDtypeStruct((NP, AW), jnp.float32),
    ],
)

_dense_call = pl.pallas_call(
    _dense_body,
    grid=(GRID,),
    in_specs=[
        pl.BlockSpec((NC, R, DH), lambda i: (0, i, 0)),
        pl.BlockSpec((R, AW), lambda i: (i, 0)),
        pl.BlockSpec((D, 2 * D), lambda i: (0, 0)),
        pl.BlockSpec((1, 2 * D), lambda i: (0, 0)),
        pl.BlockSpec((2 * D, D), lambda i: (0, 0)),
        pl.BlockSpec((D, 1), lambda i: (0, 0)),
    ],
    out_specs=pl.BlockSpec((8, R), lambda i: (0, i)),
    out_shape=jax.ShapeDtypeStruct((8, NP), jnp.float32),
)

_final_call = pl.pallas_call(
    _final_body,
    grid=(GRID,),
    in_specs=[
        pl.BlockSpec((NC, R, 1), lambda i: (0, i, 0)),
        pl.BlockSpec((R, AW), lambda i: (i, 0)),
        pl.BlockSpec((1, D), lambda i: (0, 0)),
        pl.BlockSpec((1, D), lambda i: (0, 0)),
        pl.BlockSpec((1, 1), lambda i: (0, 0)),
    ],
    out_specs=pl.BlockSpec((R, 1), lambda i: (i, 0)),
    out_shape=jax.ShapeDtypeStruct((NP, 1), jnp.float32),
)


@jax.jit
def kernel(perturbation_mask, edge_index, emb_weight, W1, b1, W2, b2,
           head_W, head_b):
  src = edge_index[0]
  dst = edge_index[1]
  npad = E_PAD - E
  src1 = jnp.concatenate([src, jnp.zeros((npad,), jnp.int32)])
  pad_dst = DUMP + (jnp.arange(npad, dtype=jnp.int32) % (NP - N))
  dst1 = jnp.concatenate([dst, pad_dst])
  src2 = src1.reshape(ECH, CH)
  dst2 = dst1.reshape(ECH, CH)
  emb_pad = jnp.pad(emb_weight, ((0, NP - N), (0, 0)))
  mask_pad = jnp.pad(perturbation_mask[:, None], ((0, NP - N), (0, 0)))

  deg2 = _deg_call(dst1)[:, :, None]
  y, dinv = _scale_call(emb_pad, mask_pad, deg2)
  z = _agg_call(src2, dst2, y)
  t8 = _dense_call(z, dinv, W1, b1[None, :], W2, head_W)
  u2 = _sagg_call(src1, dst1, t8)[:, :, None]
  out = _final_call(u2, dinv, b2[None, :], head_W[:, 0][None, :],
                    head_b[None, :])
  return out[:N, 0]
